# baseline (speedup 1.0000x reference)
import jax
import jax.numpy as jnp
from jax import lax
from jax.experimental import pallas as pl
from jax.experimental.pallas import tpu as pltpu
from jax.experimental.pallas import tpu_sc as plsc

B, S, D, E, K = 4, 8192, 1024, 8, 2
T = B * S

NC, NS, L = 2, 16, 16
NW = NC * NS

NCH = 2
TC_ = T // NCH
TW = TC_ // NW


def _logits_body(x_ref, w_ref, out_ref):
    out_ref[...] = lax.dot_general(
        w_ref[...], x_ref[...], (((1,), (1,)), ((), ())),
        preferred_element_type=jnp.float32,
    )


def _logits_tc(x, gate_w, bt):
    nb = TC_ // bt
    return pl.pallas_call(
        _logits_body,
        grid=(nb,),
        in_specs=[
            pl.BlockSpec((bt, D), lambda i: (i, 0)),
            pl.BlockSpec((E, D), lambda i: (0, 0)),
        ],
        out_specs=pl.BlockSpec((E, bt), lambda i: (0, i)),
        out_shape=jax.ShapeDtypeStruct((E, TC_), jnp.float32),
        compiler_params=pltpu.CompilerParams(
            dimension_semantics=("arbitrary",),
        ),
    )(x, gate_w)


def _route_sc_body(lg_hbm, w_hbm, i_hbm, part_hbm,
                   lg_v, w1_v, w2_v, i1_v, i2_v, part_v, sem):
    wid = lax.axis_index("s") * NC + lax.axis_index("c")
    base = wid * TW

    copies = [
        pltpu.make_async_copy(lg_hbm.at[e, pl.ds(base, TW)], lg_v.at[e], sem)
        for e in range(E)
    ]
    for c in copies:
        c.start()
    for c in copies:
        c.wait()

    zero_f = jnp.zeros((L,), jnp.float32)

    def step(j, acc):
        ls = [lg_v[e, pl.ds(j * L, L)] for e in range(E)]
        m = ls[0]
        for e in range(1, E):
            m = jnp.maximum(m, ls[e])
        exs = [jnp.exp(l - m) for l in ls]
        ssum = exs[0]
        for e in range(1, E):
            ssum = ssum + exs[e]
        rcp = 1.0 / ssum
        ps = [ex * rcp for ex in exs]

        best_v = ps[0]
        best_i = jnp.zeros((L,), jnp.int32)
        for e in range(1, E):
            gt = ps[e] > best_v
            best_v = jnp.where(gt, ps[e], best_v)
            best_i = jnp.where(gt, e, best_i)
        sec_v = zero_f - 1.0
        sec_i = jnp.zeros((L,), jnp.int32)
        for e in range(E):
            gt = (ps[e] > sec_v) & (best_i != e)
            sec_v = jnp.where(gt, ps[e], sec_v)
            sec_i = jnp.where(gt, e, sec_i)

        rden = 1.0 / (best_v + sec_v)
        w1_v[pl.ds(j * L, L)] = best_v * rden
        w2_v[pl.ds(j * L, L)] = sec_v * rden
        i1_v[pl.ds(j * L, L)] = best_i
        i2_v[pl.ds(j * L, L)] = sec_i

        new_acc = []
        for e in range(E):
            new_acc.append(acc[e] + ps[e])
        for e in range(E):
            hit = jnp.where(best_i == e, 1.0, 0.0) + jnp.where(sec_i == e, 1.0, 0.0)
            new_acc.append(acc[E + e] + hit)
        return tuple(new_acc)

    acc0 = tuple(zero_f for _ in range(2 * E))
    acc = lax.fori_loop(0, TW // L, step, acc0, unroll=2)

    for e in range(2 * E):
        part_v[e, :] = acc[e]

    pltpu.sync_copy(w1_v, w_hbm.at[0, pl.ds(base, TW)])
    pltpu.sync_copy(w2_v, w_hbm.at[1, pl.ds(base, TW)])
    pltpu.sync_copy(i1_v, i_hbm.at[0, pl.ds(base, TW)])
    pltpu.sync_copy(i2_v, i_hbm.at[1, pl.ds(base, TW)])
    pltpu.sync_copy(part_v, part_hbm.at[wid])


_route_sc = pl.kernel(
    _route_sc_body,
    out_type=[
        jax.ShapeDtypeStruct((K, TC_), jnp.float32),
        jax.ShapeDtypeStruct((K, TC_), jnp.int32),
        jax.ShapeDtypeStruct((NW, 2 * E, L), jnp.float32),
    ],
    mesh=plsc.VectorSubcoreMesh(
        core_axis_name="c", subcore_axis_name="s", num_cores=NC, num_subcores=NS
    ),
    scratch_types=[
        pltpu.VMEM((E, TW), jnp.float32),
        pltpu.VMEM((TW,), jnp.float32),
        pltpu.VMEM((TW,), jnp.float32),
        pltpu.VMEM((TW,), jnp.int32),
        pltpu.VMEM((TW,), jnp.int32),
        pltpu.VMEM((2 * E, L), jnp.float32),
        pltpu.SemaphoreType.DMA,
    ],
)


@jax.jit
def _moe_router(x, gate_w):
    wts, ids, parts = [], [], []
    for c in range(NCH):
        logits = _logits_tc(x[c * TC_:(c + 1) * TC_], gate_w, bt=4096)
        w_c, i_c, p_c = _route_sc(logits)
        wts.append(w_c)
        ids.append(i_c)
        parts.append(p_c)
    w = wts[0] if NCH == 1 else jnp.concatenate(wts, axis=1)
    i = ids[0] if NCH == 1 else jnp.concatenate(ids, axis=1)
    part = parts[0] if NCH == 1 else sum(parts[1:], parts[0])
    routing_weights = w.T.reshape(B, S, K, 1)
    selected_experts = i.T.reshape(B, S, K)
    tot = jnp.sum(part, axis=(0, 2)) / jnp.float32(T)
    aux = jnp.float32(E) * jnp.sum(tot[:E] * tot[E:])
    return routing_weights, selected_experts, aux


def kernel(hidden_states, gate_w):
    x = hidden_states.reshape(T, D)
    return _moe_router(x, gate_w)


# speedup vs baseline: 2.1847x; 2.1847x over previous
import jax
import jax.numpy as jnp
from jax import lax
from jax.experimental import pallas as pl
from jax.experimental.pallas import tpu as pltpu
from jax.experimental.pallas import tpu_sc as plsc

B, S, D, E, K = 4, 8192, 1024, 8, 2
T = B * S

NC, NS, L = 2, 16, 16
NW = NC * NS

NCH = 1
TC_ = T // NCH
TW = TC_ // NW


def _logits_body(x_ref, w_ref, out_ref):
    out_ref[...] = lax.dot_general(
        w_ref[...], x_ref[...], (((1,), (1,)), ((), ())),
        preferred_element_type=jnp.float32,
    )


def _logits_tc(x, gate_w, bt):
    nb = TC_ // bt
    return pl.pallas_call(
        _logits_body,
        grid=(nb,),
        in_specs=[
            pl.BlockSpec((bt, D), lambda i: (i, 0)),
            pl.BlockSpec((E, D), lambda i: (0, 0)),
        ],
        out_specs=pl.BlockSpec((E, bt), lambda i: (0, i)),
        out_shape=jax.ShapeDtypeStruct((E, TC_), jnp.float32),
        compiler_params=pltpu.CompilerParams(
            dimension_semantics=("arbitrary",),
        ),
    )(x, gate_w)


def _route_sc_body(lg_hbm, w_hbm, i_hbm, part_hbm,
                   lg_v, w1_v, w2_v, i1_v, i2_v, part_v, sem):
    wid = lax.axis_index("s") * NC + lax.axis_index("c")
    base = wid * TW

    copies = [
        pltpu.make_async_copy(lg_hbm.at[e, pl.ds(base, TW)], lg_v.at[e], sem)
        for e in range(E)
    ]
    for c in copies:
        c.start()
    for c in copies:
        c.wait()

    zero_f = jnp.zeros((L,), jnp.float32)

    def step(j, acc):
        ls = [lg_v[e, pl.ds(j * L, L)] for e in range(E)]
        m = ls[0]
        for e in range(1, E):
            m = jnp.maximum(m, ls[e])
        exs = [jnp.exp(l - m) for l in ls]
        ssum = exs[0]
        for e in range(1, E):
            ssum = ssum + exs[e]
        rcp = 1.0 / ssum
        ps = [ex * rcp for ex in exs]

        best_v = ps[0]
        best_i = jnp.zeros((L,), jnp.int32)
        for e in range(1, E):
            gt = ps[e] > best_v
            best_v = jnp.where(gt, ps[e], best_v)
            best_i = jnp.where(gt, e, best_i)
        sec_v = zero_f - 1.0
        sec_i = jnp.zeros((L,), jnp.int32)
        for e in range(E):
            gt = (ps[e] > sec_v) & (best_i != e)
            sec_v = jnp.where(gt, ps[e], sec_v)
            sec_i = jnp.where(gt, e, sec_i)

        rden = 1.0 / (best_v + sec_v)
        w1_v[pl.ds(j * L, L)] = best_v * rden
        w2_v[pl.ds(j * L, L)] = sec_v * rden
        i1_v[pl.ds(j * L, L)] = best_i
        i2_v[pl.ds(j * L, L)] = sec_i

        new_acc = []
        for e in range(E):
            new_acc.append(acc[e] + ps[e])
        for e in range(E):
            hit = jnp.where(best_i == e, 1.0, 0.0) + jnp.where(sec_i == e, 1.0, 0.0)
            new_acc.append(acc[E + e] + hit)
        return tuple(new_acc)

    acc0 = tuple(zero_f for _ in range(2 * E))
    acc = lax.fori_loop(0, TW // L, step, acc0, unroll=2)

    for e in range(2 * E):
        part_v[e, :] = acc[e]

    pltpu.sync_copy(w1_v, w_hbm.at[0, pl.ds(base, TW)])
    pltpu.sync_copy(w2_v, w_hbm.at[1, pl.ds(base, TW)])
    pltpu.sync_copy(i1_v, i_hbm.at[0, pl.ds(base, TW)])
    pltpu.sync_copy(i2_v, i_hbm.at[1, pl.ds(base, TW)])
    pltpu.sync_copy(part_v, part_hbm.at[wid])


_route_sc = pl.kernel(
    _route_sc_body,
    out_type=[
        jax.ShapeDtypeStruct((K, TC_), jnp.float32),
        jax.ShapeDtypeStruct((K, TC_), jnp.int32),
        jax.ShapeDtypeStruct((NW, 2 * E, L), jnp.float32),
    ],
    mesh=plsc.VectorSubcoreMesh(
        core_axis_name="c", subcore_axis_name="s", num_cores=NC, num_subcores=NS
    ),
    scratch_types=[
        pltpu.VMEM((E, TW), jnp.float32),
        pltpu.VMEM((TW,), jnp.float32),
        pltpu.VMEM((TW,), jnp.float32),
        pltpu.VMEM((TW,), jnp.int32),
        pltpu.VMEM((TW,), jnp.int32),
        pltpu.VMEM((2 * E, L), jnp.float32),
        pltpu.SemaphoreType.DMA,
    ],
)


@jax.jit
def _moe_router(x, gate_w):
    wts, ids, parts = [], [], []
    for c in range(NCH):
        logits = _logits_tc(x[c * TC_:(c + 1) * TC_], gate_w, bt=4096)
        w_c, i_c, p_c = _route_sc(logits)
        wts.append(w_c)
        ids.append(i_c)
        parts.append(p_c)
    w = wts[0] if NCH == 1 else jnp.concatenate(wts, axis=1)
    i = ids[0] if NCH == 1 else jnp.concatenate(ids, axis=1)
    part = parts[0] if NCH == 1 else sum(parts[1:], parts[0])
    routing_weights = w.T.reshape(B, S, K, 1)
    selected_experts = i.T.reshape(B, S, K)
    tot = jnp.sum(part, axis=(0, 2)) / jnp.float32(T)
    aux = jnp.float32(E) * jnp.sum(tot[:E] * tot[E:])
    return routing_weights, selected_experts, aux


def kernel(hidden_states, gate_w):
    x = hidden_states.reshape(T, D)
    return _moe_router(x, gate_w)


# speedup vs baseline: 3.5440x; 1.6222x over previous
import functools

import jax
import jax.numpy as jnp
from jax import lax
from jax.experimental import pallas as pl
from jax.experimental.pallas import tpu as pltpu

B, S, D, E, K = 4, 8192, 1024, 8, 2
T = B * S


def _router_body(x_ref, w_ref, wout_ref, sel_ref, psum_ref, cnt_ref, aux_ref):
    pid = pl.program_id(0)
    nblocks = pl.num_programs(0)

    @pl.when(pid == 0)
    def _init():
        psum_ref[...] = jnp.zeros_like(psum_ref)
        cnt_ref[...] = jnp.zeros_like(cnt_ref)
        aux_ref[...] = jnp.zeros_like(aux_ref)

    x = x_ref[...]
    w = w_ref[...]
    logits = lax.dot_general(
        w, x, (((1,), (1,)), ((), ())), preferred_element_type=jnp.float32
    )

    m = jnp.max(logits, axis=0, keepdims=True)
    ex = jnp.exp(logits - m)
    s = jnp.sum(ex, axis=0, keepdims=True)
    p = ex / s

    iota = lax.broadcasted_iota(jnp.int32, p.shape, 0)
    m1 = jnp.max(p, axis=0, keepdims=True)
    i1 = jnp.min(jnp.where(p == m1, iota, E), axis=0, keepdims=True)
    p_m = jnp.where(iota == i1, -1.0, p)
    m2 = jnp.max(p_m, axis=0, keepdims=True)
    i2 = jnp.min(jnp.where(p_m == m2, iota, E), axis=0, keepdims=True)

    denom = m1 + m2
    wout_ref[...] = jnp.concatenate([m1 / denom, m2 / denom], axis=0)
    sel_ref[...] = jnp.concatenate([i1, i2], axis=0)

    onehot = (iota == i1).astype(jnp.float32) + (iota == i2).astype(jnp.float32)
    psum_ref[...] += jnp.sum(p, axis=1, keepdims=True)
    cnt_ref[...] += jnp.sum(onehot, axis=1, keepdims=True)

    @pl.when(pid == nblocks - 1)
    def _fin():
        aux_ref[...] = (
            jnp.float32(E)
            * jnp.sum(psum_ref[...] * cnt_ref[...], keepdims=True)
            / jnp.float32(T * T)
        )[:1, :]


@functools.partial(jax.jit, static_argnames=("bt",))
def _router(x, gate_w, bt=2048):
    nb = T // bt
    wout, sel, _, _, aux = pl.pallas_call(
        _router_body,
        grid=(nb,),
        in_specs=[
            pl.BlockSpec((bt, D), lambda i: (i, 0)),
            pl.BlockSpec((E, D), lambda i: (0, 0)),
        ],
        out_specs=[
            pl.BlockSpec((K, bt), lambda i: (0, i)),
            pl.BlockSpec((K, bt), lambda i: (0, i)),
            pl.BlockSpec((E, 1), lambda i: (0, 0)),
            pl.BlockSpec((E, 1), lambda i: (0, 0)),
            pl.BlockSpec((1, 1), lambda i: (0, 0)),
        ],
        out_shape=[
            jax.ShapeDtypeStruct((K, T), jnp.float32),
            jax.ShapeDtypeStruct((K, T), jnp.int32),
            jax.ShapeDtypeStruct((E, 1), jnp.float32),
            jax.ShapeDtypeStruct((E, 1), jnp.float32),
            jax.ShapeDtypeStruct((1, 1), jnp.float32),
        ],
        compiler_params=pltpu.CompilerParams(
            dimension_semantics=("arbitrary",),
        ),
    )(x, gate_w)
    return wout, sel, aux


def kernel(hidden_states, gate_w):
    x = hidden_states.reshape(T, D)
    wout, sel, aux = _router(x, gate_w)
    routing_weights = wout.T.reshape(B, S, K, 1)
    selected_experts = sel.T.reshape(B, S, K)
    return routing_weights, selected_experts, aux.reshape(())
